# Initial kernel scaffold; baseline (speedup 1.0000x reference)
import jax
import jax.numpy as jnp
from jax.experimental import pallas as pl


def kernel(ids, id_embed, feat_tbl, W, gamma, prod_mask):
    raise NotImplementedError("write your pallas kernel here")


# speedup vs baseline: 7.4474x; 7.4474x over previous
import functools

import jax
import jax.numpy as jnp
from jax import lax
from jax.experimental import pallas as pl
from jax.experimental.pallas import tpu as pltpu
from jax.experimental.pallas import tpu_sc as plsc

_ROWS_PER_BLOCK = 1000
_CHUNK = 128


def _build_table_body(idemb_ref, feat_ref, w_ref, maskf_ref, t_ref):
    proj = lax.dot_general(
        feat_ref[...], w_ref[...],
        (((1,), (1,)), ((), ())),
        preferred_element_type=jnp.float32,
    )
    t_ref[...] = idemb_ref[...] + maskf_ref[...] * proj


def _build_table(id_embed, feat_tbl, w_gamma, maskf):
    v, d = id_embed.shape
    rb = _ROWS_PER_BLOCK
    return pl.pallas_call(
        _build_table_body,
        grid=(v // rb,),
        in_specs=[
            pl.BlockSpec((rb, d), lambda i: (i, 0)),
            pl.BlockSpec((rb, d), lambda i: (i, 0)),
            pl.BlockSpec(w_gamma.shape, lambda i: (0, 0)),
            pl.BlockSpec((rb, 1), lambda i: (i, 0)),
        ],
        out_specs=pl.BlockSpec((rb, d), lambda i: (i, 0)),
        out_shape=jax.ShapeDtypeStruct((v, d), jnp.float32),
    )(id_embed, feat_tbl, w_gamma, maskf)


@functools.cache
def _make_gather(v, d, b):
    info = plsc.get_sparse_core_info()
    nc, ns = info.num_cores, info.num_subcores
    nw = nc * ns
    chunk = _CHUNK
    n_chunks = b // (nw * chunk)
    per_worker = n_chunks * chunk
    mesh = plsc.VectorSubcoreMesh(core_axis_name="c", subcore_axis_name="s")

    @functools.partial(
        pl.kernel,
        mesh=mesh,
        compiler_params=pltpu.CompilerParams(use_tc_tiling_on_sc=False),
        out_type=jax.ShapeDtypeStruct((b, d), jnp.float32),
        scratch_types=[
            pltpu.VMEM((n_chunks, chunk), jnp.int32),
            pltpu.VMEM((chunk, d), jnp.float32),
            pltpu.SemaphoreType.DMA,
        ],
    )
    def gather_kernel(table_hbm, idx_hbm, out_hbm, idx_v, rows_v, sem):
        wid = lax.axis_index("s") * nc + lax.axis_index("c")
        pltpu.sync_copy(idx_hbm.at[wid], idx_v)

        def body(j, carry):
            pltpu.async_copy(table_hbm.at[idx_v.at[j]], rows_v, sem).wait()
            pltpu.sync_copy(
                rows_v, out_hbm.at[pl.ds(wid * per_worker + j * chunk, chunk)]
            )
            return carry

        lax.fori_loop(0, n_chunks, body, 0)

    return gather_kernel, nw, n_chunks, chunk


def kernel(ids, id_embed, feat_tbl, W, gamma, prod_mask):
    v, d = id_embed.shape
    bsz, hist = ids.shape
    b = bsz * hist

    maskf = prod_mask.astype(jnp.float32)[:, None]
    w_gamma = W * gamma.astype(jnp.float32)
    table = _build_table(id_embed, feat_tbl, w_gamma, maskf)

    gather_fn, nw, n_chunks, chunk = _make_gather(v, d, b)
    idx = ids.astype(jnp.int32).reshape(nw, n_chunks, chunk)
    out = gather_fn(table, idx)
    return out.reshape(bsz, hist, d)


# speedup vs baseline: 8.0013x; 1.0744x over previous
import functools

import jax
import jax.numpy as jnp
from jax import lax
from jax.experimental import pallas as pl
from jax.experimental.pallas import tpu as pltpu
from jax.experimental.pallas import tpu_sc as plsc

_CHUNK = 128
_NB = 128
_K = 64


@functools.cache
def _make_gather(v, d, b):
    info = plsc.get_sparse_core_info()
    nc, ns = info.num_cores, info.num_subcores
    nw = nc * ns
    chunk = _CHUNK
    n_chunks = b // (nw * chunk)
    per_worker = n_chunks * chunk
    mesh = plsc.VectorSubcoreMesh(core_axis_name="c", subcore_axis_name="s")

    @functools.partial(
        pl.kernel,
        mesh=mesh,
        compiler_params=pltpu.CompilerParams(use_tc_tiling_on_sc=False),
        out_type=jax.ShapeDtypeStruct((b, d), jnp.float32),
        scratch_types=[
            pltpu.VMEM((n_chunks, chunk), jnp.int32),
            pltpu.VMEM((chunk, d), jnp.float32),
            pltpu.SemaphoreType.DMA,
        ],
    )
    def gather_kernel(table_hbm, idx_hbm, out_hbm, idx_v, rows_v, sem):
        wid = lax.axis_index("s") * nc + lax.axis_index("c")
        pltpu.sync_copy(idx_hbm.at[wid], idx_v)

        def body(j, carry):
            pltpu.async_copy(table_hbm.at[idx_v.at[j]], rows_v, sem).wait()
            pltpu.sync_copy(
                rows_v, out_hbm.at[pl.ds(wid * per_worker + j * chunk, chunk)]
            )
            return carry

        lax.fori_loop(0, n_chunks, body, 0)

    return gather_kernel, nw, n_chunks, chunk


def _assemble_body(g_ref, ids_ref, pids_ref, featp_ref, wg_ref, out_ref, tmp_ref):
    nb, hist = ids_ref.shape
    d = wg_ref.shape[0]
    corr_tbl = lax.dot_general(
        featp_ref[...], wg_ref[...],
        (((1,), (1,)), ((), ())),
        preferred_element_type=jnp.float32,
        precision=lax.Precision.HIGHEST,
    )
    ids_b = ids_ref[...]
    onehot = (ids_b[:, :, None] == pids_ref[...][0][None, None, :]).astype(
        jnp.float32
    )
    corr = lax.dot_general(
        onehot, corr_tbl,
        (((2,), (0,)), ((), ())),
        preferred_element_type=jnp.float32,
        precision=lax.Precision.HIGHEST,
    )
    npairs = nb * hist // 2
    tmp_ref[...] = g_ref[...].reshape(npairs, 2, d)
    out_ref[...] = tmp_ref[...].reshape(nb, hist, d) + corr


def _assemble(g_view, ids, pids, featp, wg, bsz, hist, d):
    nb = _NB
    k = _K
    rows_per_blk = nb * hist * d // 128
    return pl.pallas_call(
        _assemble_body,
        grid=(bsz // nb,),
        in_specs=[
            pl.BlockSpec((rows_per_blk, 128), lambda i: (i, 0)),
            pl.BlockSpec((nb, hist), lambda i: (i, 0)),
            pl.BlockSpec((1, k), lambda i: (0, 0)),
            pl.BlockSpec((k, d), lambda i: (0, 0)),
            pl.BlockSpec((d, d), lambda i: (0, 0)),
        ],
        out_specs=pl.BlockSpec((nb, hist, d), lambda i: (i, 0, 0)),
        out_shape=jax.ShapeDtypeStruct((bsz, hist, d), jnp.float32),
        scratch_shapes=[pltpu.VMEM((nb * hist // 2, 2, d), jnp.float32)],
    )(g_view, ids, pids, featp, wg)


def kernel(ids, id_embed, feat_tbl, W, gamma, prod_mask):
    v, d = id_embed.shape
    bsz, hist = ids.shape
    b = bsz * hist

    pidx = jnp.nonzero(prod_mask, size=_K, fill_value=0)[0].astype(jnp.int32)
    count = jnp.sum(prod_mask.astype(jnp.int32))
    pids = jnp.where(jnp.arange(_K, dtype=jnp.int32) < count, pidx, -1)
    pids = pids.reshape(1, _K)
    featp = jnp.take(feat_tbl, pidx, axis=0)
    wg = W * gamma.astype(jnp.float32)

    gather_fn, nw, n_chunks, chunk = _make_gather(v, d, b)
    idsx = ids.astype(jnp.int32).reshape(nw, n_chunks, chunk)
    gathered = gather_fn(id_embed, idsx)
    g_view = jnp.reshape(gathered, (b * d // 128, 128))

    return _assemble(g_view, ids.astype(jnp.int32), pids, featp, wg, bsz, hist, d)


# speedup vs baseline: 13.9650x; 1.7453x over previous
import functools

import jax
import jax.numpy as jnp
from jax import lax
from jax.experimental import pallas as pl
from jax.experimental.pallas import tpu as pltpu
from jax.experimental.pallas import tpu_sc as plsc

_CHUNK = 128
_K = 64


@functools.cache
def _make_gather(v, d, bsz, hist):
    info = plsc.get_sparse_core_info()
    nc, ns = info.num_cores, info.num_subcores
    nw = nc * ns
    chunk = _CHUNK
    lanes_per_worker = bsz // nw
    assert lanes_per_worker == chunk
    mesh = plsc.VectorSubcoreMesh(core_axis_name="c", subcore_axis_name="s")

    @functools.partial(
        pl.kernel,
        mesh=mesh,
        compiler_params=pltpu.CompilerParams(use_tc_tiling_on_sc=False),
        out_type=jax.ShapeDtypeStruct((2 * bsz * hist, d), jnp.float32),
        scratch_types=[
            pltpu.VMEM((hist, chunk), jnp.int32),
            pltpu.VMEM((chunk, d), jnp.float32),
            pltpu.VMEM((1, chunk), jnp.int32),
            pltpu.SemaphoreType.DMA,
            pltpu.SemaphoreType.DMA,
        ],
    )
    def gather_kernel(table_hbm, idst_hbm, out_hbm, idx_v, rows_v, sidx_v, gsem, ssem):
        wid = lax.axis_index("s") * nc + lax.axis_index("c")
        col0 = wid * chunk
        pltpu.sync_copy(idst_hbm.at[:, pl.ds(col0, chunk)], idx_v)

        def body(j, carry):
            pltpu.async_copy(table_hbm.at[idx_v.at[j]], rows_v, gsem).wait()
            base = 2 * (j * bsz + col0)
            for k in range(chunk // 16):
                sidx_v[0, pl.ds(k * 16, 16)] = (
                    lax.iota(jnp.int32, 16) * 2 + (base + 32 * k)
                )
            pltpu.async_copy(rows_v, out_hbm.at[sidx_v.at[0]], ssem).wait()
            return carry

        lax.fori_loop(0, hist, body, 0)

    return gather_kernel


def _assemble_body(g_ref, idst_ref, pids_ref, featp_ref, wg_ref, out_ref):
    corr_tbl = lax.dot_general(
        featp_ref[...], wg_ref[...],
        (((1,), (1,)), ((), ())),
        preferred_element_type=jnp.float32,
        precision=lax.Precision.HIGHEST,
    )
    ids_row = idst_ref[pl.ds(pl.program_id(0), 1), :]
    pids_t = pids_ref[...].T
    onehot = (pids_t == ids_row).astype(jnp.float32)
    corr_t = lax.dot_general(
        corr_tbl, onehot,
        (((0,), (0,)), ((), ())),
        preferred_element_type=jnp.float32,
        precision=lax.Precision.HIGHEST,
    )
    d = featp_ref.shape[1]
    g_t = g_ref[...][:, :d].T
    out_ref[0, :, :] = g_t + corr_t


def _assemble(g_view, idst, pids, featp, wg, bsz, hist, d):
    return pl.pallas_call(
        _assemble_body,
        grid=(hist,),
        in_specs=[
            pl.BlockSpec((bsz, 2 * d), lambda h: (h, 0)),
            pl.BlockSpec((hist, bsz), lambda h: (0, 0)),
            pl.BlockSpec((1, _K), lambda h: (0, 0)),
            pl.BlockSpec((_K, d), lambda h: (0, 0)),
            pl.BlockSpec((d, d), lambda h: (0, 0)),
        ],
        out_specs=pl.BlockSpec((1, d, bsz), lambda h: (h, 0, 0)),
        out_shape=jax.ShapeDtypeStruct((hist, d, bsz), jnp.float32),
    )(g_view, idst, pids, featp, wg)


def kernel(ids, id_embed, feat_tbl, W, gamma, prod_mask):
    v, d = id_embed.shape
    bsz, hist = ids.shape

    pidx = jnp.nonzero(prod_mask, size=_K, fill_value=0)[0].astype(jnp.int32)
    count = jnp.sum(prod_mask.astype(jnp.int32))
    pids = jnp.where(jnp.arange(_K, dtype=jnp.int32) < count, pidx, -1)
    pids = pids.reshape(1, _K)
    featp = jnp.take(feat_tbl, pidx, axis=0)
    wg = W * gamma.astype(jnp.float32)

    idst = ids.astype(jnp.int32).T

    gather_fn = _make_gather(v, d, bsz, hist)
    scat = gather_fn(id_embed, idst)
    g_view = jnp.reshape(scat, (bsz * hist, 2 * d))

    out_t = _assemble(g_view, idst, pids, featp, wg, bsz, hist, d)
    return jnp.transpose(out_t, (2, 0, 1))


# speedup vs baseline: 15.1414x; 1.0842x over previous
import functools

import jax
import jax.numpy as jnp
from jax import lax
from jax.experimental import pallas as pl
from jax.experimental.pallas import tpu as pltpu
from jax.experimental.pallas import tpu_sc as plsc

_CHUNK = 128
_K = 64


@functools.cache
def _make_gather(v, d, bsz, hist):
    info = plsc.get_sparse_core_info()
    nc, ns = info.num_cores, info.num_subcores
    nw = nc * ns
    chunk = _CHUNK
    lanes_per_worker = bsz // nw
    assert lanes_per_worker == chunk
    mesh = plsc.VectorSubcoreMesh(core_axis_name="c", subcore_axis_name="s")

    @functools.partial(
        pl.kernel,
        mesh=mesh,
        compiler_params=pltpu.CompilerParams(use_tc_tiling_on_sc=False),
        out_type=jax.ShapeDtypeStruct((2 * bsz * hist, d), jnp.float32),
        scratch_types=[
            pltpu.VMEM((hist, chunk), jnp.int32),
            pltpu.VMEM((chunk, d), jnp.float32),
            pltpu.VMEM((chunk, d), jnp.float32),
            pltpu.VMEM((1, chunk), jnp.int32),
            pltpu.VMEM((1, chunk), jnp.int32),
            pltpu.SemaphoreType.DMA,
            pltpu.SemaphoreType.DMA,
            pltpu.SemaphoreType.DMA,
            pltpu.SemaphoreType.DMA,
        ],
    )
    def gather_kernel(
        table_hbm, idst_hbm, out_hbm,
        idx_v, rows_a, rows_b, sidx_a, sidx_b, gsem_a, gsem_b, ssem_a, ssem_b,
    ):
        wid = lax.axis_index("s") * nc + lax.axis_index("c")
        col0 = wid * chunk
        pltpu.sync_copy(idst_hbm.at[:, pl.ds(col0, chunk)], idx_v)

        def fill_sidx(sidx, j):
            base = 2 * (j * bsz + col0)
            for k in range(chunk // 16):
                sidx[0, pl.ds(k * 16, 16)] = (
                    lax.iota(jnp.int32, 16) * 2 + (base + 32 * k)
                )

        def body(j2, carry):
            c0 = 2 * j2
            c1 = c0 + 1
            pltpu.async_copy(table_hbm.at[idx_v.at[c0]], rows_a, gsem_a)
            pltpu.async_copy(table_hbm.at[idx_v.at[c1]], rows_b, gsem_b)
            fill_sidx(sidx_a, c0)
            fill_sidx(sidx_b, c1)
            pltpu.make_async_copy(table_hbm.at[idx_v.at[c0]], rows_a, gsem_a).wait()
            sca = pltpu.async_copy(rows_a, out_hbm.at[sidx_a.at[0]], ssem_a)
            pltpu.make_async_copy(table_hbm.at[idx_v.at[c1]], rows_b, gsem_b).wait()
            scb = pltpu.async_copy(rows_b, out_hbm.at[sidx_b.at[0]], ssem_b)
            sca.wait()
            scb.wait()
            return carry

        lax.fori_loop(0, hist // 2, body, 0)

    return gather_kernel


def _assemble_body(g_ref, idst_ref, pids_ref, featp_ref, wg_ref, out_ref):
    corr_tbl = lax.dot_general(
        featp_ref[...], wg_ref[...],
        (((1,), (1,)), ((), ())),
        preferred_element_type=jnp.float32,
        precision=lax.Precision.HIGHEST,
    )
    ids_row = idst_ref[pl.ds(pl.program_id(0), 1), :]
    pids_t = pids_ref[...].T
    onehot = (pids_t == ids_row).astype(jnp.float32)
    corr_t = lax.dot_general(
        corr_tbl, onehot,
        (((0,), (0,)), ((), ())),
        preferred_element_type=jnp.float32,
        precision=lax.Precision.HIGHEST,
    )
    d = featp_ref.shape[1]
    g_t = g_ref[...][:, :d].T
    out_ref[0, :, :] = g_t + corr_t


def _assemble(g_view, idst, pids, featp, wg, bsz, hist, d):
    return pl.pallas_call(
        _assemble_body,
        grid=(hist,),
        in_specs=[
            pl.BlockSpec((bsz, 2 * d), lambda h: (h, 0)),
            pl.BlockSpec((hist, bsz), lambda h: (0, 0)),
            pl.BlockSpec((1, _K), lambda h: (0, 0)),
            pl.BlockSpec((_K, d), lambda h: (0, 0)),
            pl.BlockSpec((d, d), lambda h: (0, 0)),
        ],
        out_specs=pl.BlockSpec((1, d, bsz), lambda h: (h, 0, 0)),
        out_shape=jax.ShapeDtypeStruct((hist, d, bsz), jnp.float32),
    )(g_view, idst, pids, featp, wg)


def kernel(ids, id_embed, feat_tbl, W, gamma, prod_mask):
    v, d = id_embed.shape
    bsz, hist = ids.shape

    pidx = jnp.nonzero(prod_mask, size=_K, fill_value=0)[0].astype(jnp.int32)
    count = jnp.sum(prod_mask.astype(jnp.int32))
    pids = jnp.where(jnp.arange(_K, dtype=jnp.int32) < count, pidx, -1)
    pids = pids.reshape(1, _K)
    featp = jnp.take(feat_tbl, pidx, axis=0)
    wg = W * gamma.astype(jnp.float32)

    idst = ids.astype(jnp.int32).T

    gather_fn = _make_gather(v, d, bsz, hist)
    scat = gather_fn(id_embed, idst)
    g_view = jnp.reshape(scat, (bsz * hist, 2 * d))

    out_t = _assemble(g_view, idst, pids, featp, wg, bsz, hist, d)
    return jnp.transpose(out_t, (2, 0, 1))
